# baseline (speedup 1.0000x reference)
import functools

import jax
import jax.numpy as jnp
from jax import lax
from jax.experimental import pallas as pl
from jax.experimental.pallas import tpu as pltpu
from jax.experimental.pallas import tpu_sc as plsc

N_USERS = 1024
N_ITEMS = 100000
N_TEST = 200

IBLK = 2048

_SC_INFO = plsc.get_sparse_core_info()
_NC = _SC_INFO.num_cores
_NS = _SC_INFO.num_subcores
_NW = _NC * _NS
_L = _SC_INFO.num_lanes

_TOTAL_IDX = N_USERS * N_TEST
_IDX_PER_W = _TOTAL_IDX // _NW


def _sum_body(train_ref, out_ref):
    ones = jnp.ones((1, N_USERS), dtype=jnp.float32)
    out_ref[...] = jax.lax.dot_general(
        ones, train_ref[...], (((1,), (1,)), ((), ())),
        preferred_element_type=jnp.float32,
    )


def _popularity_sum(train):
    train_t = train.T
    grid = pl.cdiv(N_ITEMS, IBLK)
    score2d = pl.pallas_call(
        _sum_body,
        grid=(grid,),
        in_specs=[pl.BlockSpec((IBLK, N_USERS), lambda i: (i, 0))],
        out_specs=pl.BlockSpec((1, IBLK), lambda i: (0, i)),
        out_shape=jax.ShapeDtypeStruct((1, N_ITEMS), jnp.float32),
    )(train_t)
    return score2d.reshape(N_ITEMS)


@functools.partial(
    pl.kernel,
    out_type=jax.ShapeDtypeStruct((_TOTAL_IDX,), jnp.float32),
    mesh=plsc.VectorSubcoreMesh(core_axis_name="c", subcore_axis_name="s"),
    scratch_types=[
        pltpu.VMEM((_IDX_PER_W,), jnp.int32),
        pltpu.VMEM((_IDX_PER_W,), jnp.float32),
        pltpu.SemaphoreType.DMA,
    ],
)
def _gather_kernel(score_hbm, idx_hbm, out_hbm, idx_v, out_v, sem):
    wid = lax.axis_index("s") * _NC + lax.axis_index("c")
    base = wid * _IDX_PER_W
    pltpu.sync_copy(idx_hbm.at[pl.ds(base, _IDX_PER_W)], idx_v)
    pltpu.async_copy(score_hbm.at[idx_v], out_v, sem).wait()
    pltpu.sync_copy(out_v, out_hbm.at[pl.ds(base, _IDX_PER_W)])


def kernel(train, test_items):
    score = _popularity_sum(train)
    idx = test_items.T.reshape(-1).astype(jnp.int32)
    flat = _gather_kernel(score, idx)
    return flat.reshape(N_TEST, N_USERS).T


# speedup vs baseline: 1.0118x; 1.0118x over previous
import functools

import jax
import jax.numpy as jnp
from jax import lax
from jax.experimental import pallas as pl
from jax.experimental.pallas import tpu as pltpu
from jax.experimental.pallas import tpu_sc as plsc

N_USERS = 1024
N_ITEMS = 100000
N_TEST = 200

IBLK = 2560

_SC_INFO = plsc.get_sparse_core_info()
_NC = _SC_INFO.num_cores
_NS = _SC_INFO.num_subcores
_NW = _NC * _NS
_L = _SC_INFO.num_lanes

_TOTAL_IDX = N_USERS * N_TEST
_IDX_PER_W = _TOTAL_IDX // _NW


def _sum_body(train_ref, out_ref):
    ones = jnp.ones((1, N_USERS), dtype=jnp.float32)
    out_ref[...] = jax.lax.dot_general(
        ones, train_ref[...], (((1,), (1,)), ((), ())),
        preferred_element_type=jnp.float32,
    )


def _popularity_sum(train):
    train_t = train.T
    grid = pl.cdiv(N_ITEMS, IBLK)
    score2d = pl.pallas_call(
        _sum_body,
        grid=(grid,),
        in_specs=[pl.BlockSpec((IBLK, N_USERS), lambda i: (i, 0))],
        out_specs=pl.BlockSpec((1, IBLK), lambda i: (0, i)),
        out_shape=jax.ShapeDtypeStruct((1, N_ITEMS), jnp.float32),
    )(train_t)
    return score2d.reshape(N_ITEMS)


@functools.partial(
    pl.kernel,
    out_type=jax.ShapeDtypeStruct((_TOTAL_IDX,), jnp.float32),
    mesh=plsc.VectorSubcoreMesh(core_axis_name="c", subcore_axis_name="s"),
    scratch_types=[
        pltpu.VMEM((_IDX_PER_W,), jnp.int32),
        pltpu.VMEM((_IDX_PER_W,), jnp.float32),
        pltpu.SemaphoreType.DMA,
    ],
)
def _gather_kernel(score_hbm, idx_hbm, out_hbm, idx_v, out_v, sem):
    wid = lax.axis_index("s") * _NC + lax.axis_index("c")
    base = wid * _IDX_PER_W
    pltpu.sync_copy(idx_hbm.at[pl.ds(base, _IDX_PER_W)], idx_v)
    pltpu.async_copy(score_hbm.at[idx_v], out_v, sem).wait()
    pltpu.sync_copy(out_v, out_hbm.at[pl.ds(base, _IDX_PER_W)])


def kernel(train, test_items):
    score = _popularity_sum(train)
    idx = test_items.T.reshape(-1).astype(jnp.int32)
    flat = _gather_kernel(score, idx)
    return flat.reshape(N_TEST, N_USERS).T


# speedup vs baseline: 1.0164x; 1.0046x over previous
import functools

import jax
import jax.numpy as jnp
from jax import lax
from jax.experimental import pallas as pl
from jax.experimental.pallas import tpu as pltpu
from jax.experimental.pallas import tpu_sc as plsc

N_USERS = 1024
N_ITEMS = 100000
N_TEST = 200

IBLK = 2048

_SC_INFO = plsc.get_sparse_core_info()
_NC = _SC_INFO.num_cores
_NS = _SC_INFO.num_subcores
_NW = _NC * _NS
_L = _SC_INFO.num_lanes

_TOTAL_IDX = N_USERS * N_TEST
_IDX_PER_W = _TOTAL_IDX // _NW


def _sum_body(train_ref, out_ref):
    ones = jnp.ones((1, N_USERS), dtype=jnp.float32)
    out_ref[...] = jax.lax.dot_general(
        ones, train_ref[...], (((1,), (1,)), ((), ())),
        preferred_element_type=jnp.float32,
    )


def _popularity_sum(train):
    train_t = train.T
    grid = pl.cdiv(N_ITEMS, IBLK)
    score2d = pl.pallas_call(
        _sum_body,
        grid=(grid,),
        in_specs=[pl.BlockSpec((IBLK, N_USERS), lambda i: (i, 0))],
        out_specs=pl.BlockSpec((1, IBLK), lambda i: (0, i)),
        out_shape=jax.ShapeDtypeStruct((1, N_ITEMS), jnp.float32),
    )(train_t)
    return score2d.reshape(N_ITEMS)


@functools.partial(
    pl.kernel,
    out_type=jax.ShapeDtypeStruct((_TOTAL_IDX,), jnp.float32),
    mesh=plsc.VectorSubcoreMesh(core_axis_name="c", subcore_axis_name="s"),
    scratch_types=[
        pltpu.VMEM((_IDX_PER_W,), jnp.int32),
        pltpu.VMEM((_IDX_PER_W,), jnp.float32),
        pltpu.SemaphoreType.DMA,
    ],
)
def _gather_kernel(score_hbm, idx_hbm, out_hbm, idx_v, out_v, sem):
    wid = lax.axis_index("s") * _NC + lax.axis_index("c")
    base = wid * _IDX_PER_W
    pltpu.sync_copy(idx_hbm.at[pl.ds(base, _IDX_PER_W)], idx_v)
    pltpu.async_copy(score_hbm.at[idx_v], out_v, sem).wait()
    pltpu.sync_copy(out_v, out_hbm.at[pl.ds(base, _IDX_PER_W)])


def kernel(train, test_items):
    score = _popularity_sum(train)
    idx = test_items.T.reshape(-1).astype(jnp.int32)
    flat = _gather_kernel(score, idx)
    return flat.reshape(N_TEST, N_USERS).T
